# baseline (speedup 1.0000x reference)
import functools

import jax
import jax.numpy as jnp
from jax import lax
from jax.experimental import pallas as pl
from jax.experimental.pallas import tpu as pltpu
from jax.experimental.pallas import tpu_sc as plsc

NC = 2
NS = 16
NW = NC * NS

CHUNK = 128
NBUF = 4
PRIME = 3


def _make_sc_gather(total, d):
    per_w = total // NW
    nchunks = per_w // CHUNK
    ngroups = nchunks // NBUF
    mesh = plsc.VectorSubcoreMesh(core_axis_name="c", subcore_axis_name="s")

    @functools.partial(
        pl.kernel,
        mesh=mesh,
        out_type=jax.ShapeDtypeStruct((total, d), jnp.float32),
        compiler_params=pltpu.CompilerParams(use_tc_tiling_on_sc=False, needs_layout_passes=False),
        scratch_types=[
            pltpu.VMEM((nchunks, CHUNK), jnp.int32),
            pltpu.VMEM((NBUF, CHUNK, d // 2), jnp.int32),
            pltpu.VMEM((NBUF, CHUNK, d), jnp.float32),
        ]
        + [pltpu.SemaphoreType.DMA] * (2 * NBUF),
    )
    def gather_kernel(idx_hbm, table_hbm, out_hbm, idx_v, raw_v, rows_v, *sems):
        gsems = sems[:NBUF]
        wsems = sems[NBUF:]
        wid = lax.axis_index("s") * NC + lax.axis_index("c")
        base = wid * per_w
        ie = 2 * lax.iota(jnp.int32, 16)
        io = ie + 1
        pltpu.sync_copy(idx_hbm.at[wid], idx_v)

        for b in range(PRIME):
            pltpu.async_copy(table_hbm.at[idx_v.at[b]], raw_v.at[b], gsems[b])

        def group(jo, carry):
            for b in range(NBUF):
                j = jo * NBUF + b
                pltpu.make_async_copy(
                    table_hbm.at[idx_v.at[j]], raw_v.at[b], gsems[b]
                ).wait()

                jn = j + PRIME
                bn = (b + PRIME) % NBUF

                @pl.when(jn < nchunks)
                def _():
                    pltpu.async_copy(
                        table_hbm.at[idx_v.at[jn]], raw_v.at[bn], gsems[bn]
                    )

                @pl.when(j >= NBUF)
                def _():
                    pltpu.make_async_copy(
                        rows_v.at[b],
                        out_hbm.at[pl.ds(base, CHUNK)],
                        wsems[b],
                    ).wait()

                @plsc.parallel_loop(0, CHUNK, unroll=4)
                def _(r):
                    for c in range(4):
                        w = raw_v[b, r, pl.ds(16 * c, 16)]
                        lo = lax.bitcast_convert_type(w << 16, jnp.float32)
                        hi = lax.bitcast_convert_type(
                            w & jnp.int32(-65536), jnp.float32
                        )
                        plsc.store_scatter(rows_v.at[b, r], [ie + 32 * c], lo)
                        plsc.store_scatter(rows_v.at[b, r], [io + 32 * c], hi)

                pltpu.async_copy(
                    rows_v.at[b],
                    out_hbm.at[pl.ds(base + j * CHUNK, CHUNK)],
                    wsems[b],
                )

            return carry

        lax.fori_loop(0, ngroups, group, 0)

        for b in range(NBUF):
            pltpu.make_async_copy(
                rows_v.at[b], out_hbm.at[pl.ds(base, CHUNK)], wsems[b]
            ).wait()

    return gather_kernel


def kernel(x, table):
    total = x.shape[0] * x.shape[1]
    d = table.shape[1]
    tbl = lax.bitcast_convert_type(
        table.astype(jnp.bfloat16).reshape(-1, d // 2, 2), jnp.int32
    )
    idx = x.astype(jnp.int32).reshape(NW, total // (NW * CHUNK), CHUNK)
    out = _make_sc_gather(total, d)(idx, tbl)
    return out.reshape(x.shape[0], x.shape[1], d)


# speedup vs baseline: 2.2940x; 2.2940x over previous
import functools

import jax
import jax.numpy as jnp
from jax import lax
from jax.experimental import pallas as pl
from jax.experimental.pallas import tpu as pltpu
from jax.experimental.pallas import tpu_sc as plsc

NC = 2
NS = 16
NW = NC * NS

CHUNK = 64
NBUF = 8
PRIME = 4


def _make_sc_gather(total, d):
    per_w = total // NW
    nchunks = per_w // CHUNK
    ngroups = nchunks // NBUF
    mesh = plsc.VectorSubcoreMesh(core_axis_name="c", subcore_axis_name="s")

    @functools.partial(
        pl.kernel,
        mesh=mesh,
        out_type=jax.ShapeDtypeStruct((total, d), jnp.float32),
        scratch_types=[
            pltpu.VMEM((nchunks, CHUNK), jnp.int32),
            pltpu.VMEM((NBUF, CHUNK, d), jnp.float32),
        ]
        + [pltpu.SemaphoreType.DMA] * (2 * NBUF),
    )
    def gather_kernel(idx_hbm, table_hbm, out_hbm, idx_v, rows_v, *sems):
        gsems = sems[:NBUF]
        wsems = sems[NBUF:]
        wid = lax.axis_index("s") * NC + lax.axis_index("c")
        base = wid * per_w
        pltpu.sync_copy(idx_hbm.at[wid], idx_v)

        for b in range(PRIME):
            pltpu.async_copy(table_hbm.at[idx_v.at[b]], rows_v.at[b], gsems[b])

        def group(jo, carry):
            for b in range(NBUF):
                j = jo * NBUF + b
                pltpu.make_async_copy(
                    table_hbm.at[idx_v.at[j]], rows_v.at[b], gsems[b]
                ).wait()
                pltpu.async_copy(
                    rows_v.at[b],
                    out_hbm.at[pl.ds(base + j * CHUNK, CHUNK)],
                    wsems[b],
                )
                jn = j + PRIME
                bn = (b + PRIME) % NBUF

                @pl.when(jn < nchunks)
                def _():
                    @pl.when(j >= NBUF - PRIME)
                    def _():
                        pltpu.make_async_copy(
                            rows_v.at[bn],
                            out_hbm.at[pl.ds(base, CHUNK)],
                            wsems[bn],
                        ).wait()

                    pltpu.async_copy(
                        table_hbm.at[idx_v.at[jn]], rows_v.at[bn], gsems[bn]
                    )

            return carry

        lax.fori_loop(0, ngroups, group, 0)

        for b in range(NBUF):
            pltpu.make_async_copy(
                rows_v.at[b], out_hbm.at[pl.ds(base, CHUNK)], wsems[b]
            ).wait()

    return gather_kernel


def kernel(x, table):
    total = x.shape[0] * x.shape[1]
    d = table.shape[1]
    idx = x.astype(jnp.int32).reshape(NW, total // (NW * CHUNK), CHUNK)
    out = _make_sc_gather(total, d)(idx, table)
    return out.reshape(x.shape[0], x.shape[1], d)


# speedup vs baseline: 2.3735x; 1.0346x over previous
import functools

import jax
import jax.numpy as jnp
from jax import lax
from jax.experimental import pallas as pl
from jax.experimental.pallas import tpu as pltpu
from jax.experimental.pallas import tpu_sc as plsc

NC = 2
NS = 16
NW = NC * NS

CHUNK = 128
NBUF = 5
PRIME = 3


def _make_sc_gather(total, d):
    per_w = total // NW
    nchunks = per_w // CHUNK
    ngroups = nchunks // NBUF
    mesh = plsc.VectorSubcoreMesh(core_axis_name="c", subcore_axis_name="s")

    @functools.partial(
        pl.kernel,
        mesh=mesh,
        out_type=jax.ShapeDtypeStruct((total, d), jnp.float32),
        scratch_types=[
            pltpu.VMEM((nchunks, CHUNK), jnp.int32),
            pltpu.VMEM((NBUF, CHUNK, d), jnp.float32),
            pltpu.VMEM_SHARED((NS, CHUNK, d), jnp.float32),
        ]
        + [pltpu.SemaphoreType.DMA] * (2 * NBUF + 1),
    )
    def gather_kernel(idx_hbm, table_hbm, out_hbm, idx_v, rows_v, shr_v, *sems):
        gsems = sems[:NBUF]
        wsems = sems[NBUF : 2 * NBUF]
        w2sems = sems[2 * NBUF :]
        sid = lax.axis_index("s")
        wid = sid * NC + lax.axis_index("c")
        base = wid * per_w
        pltpu.sync_copy(idx_hbm.at[wid], idx_v)

        for b in range(PRIME):
            pltpu.async_copy(table_hbm.at[idx_v.at[b]], rows_v.at[b], gsems[b])

        def group(jo, carry):
            for b in range(NBUF):
                j = jo * NBUF + b
                pltpu.make_async_copy(
                    table_hbm.at[idx_v.at[j]], rows_v.at[b], gsems[b]
                ).wait()
                if b % 2 == 0:
                    pltpu.async_copy(
                        rows_v.at[b],
                        out_hbm.at[pl.ds(base + j * CHUNK, CHUNK)],
                        wsems[b],
                    )
                else:
                    @pl.when(j >= 3)
                    def _():
                        pltpu.make_async_copy(
                            shr_v.at[sid],
                            out_hbm.at[pl.ds(base, CHUNK)],
                            w2sems[0],
                        ).wait()

                    pltpu.sync_copy(rows_v.at[b], shr_v.at[sid])
                    pltpu.async_copy(
                        shr_v.at[sid],
                        out_hbm.at[pl.ds(base + j * CHUNK, CHUNK)],
                        w2sems[0],
                    )
                jn = j + PRIME
                bn = (b + PRIME) % NBUF

                @pl.when(jn < nchunks)
                def _():
                    if bn % 2 == 0:
                        @pl.when(j >= NBUF - PRIME)
                        def _():
                            pltpu.make_async_copy(
                                rows_v.at[bn],
                                out_hbm.at[pl.ds(base, CHUNK)],
                                wsems[bn],
                            ).wait()

                    pltpu.async_copy(
                        table_hbm.at[idx_v.at[jn]], rows_v.at[bn], gsems[bn]
                    )

            return carry

        lax.fori_loop(0, ngroups, group, 0)

        for b in range(NBUF):
            if b % 2 == 0:
                pltpu.make_async_copy(
                    rows_v.at[b], out_hbm.at[pl.ds(base, CHUNK)], wsems[b]
                ).wait()
        pltpu.make_async_copy(
            shr_v.at[sid], out_hbm.at[pl.ds(base, CHUNK)], w2sems[0]
        ).wait()

    return gather_kernel


def kernel(x, table):
    total = x.shape[0] * x.shape[1]
    d = table.shape[1]
    idx = x.astype(jnp.int32).reshape(NW, total // (NW * CHUNK), CHUNK)
    out = _make_sc_gather(total, d)(idx, table)
    return out.reshape(x.shape[0], x.shape[1], d)
